# Initial kernel scaffold; baseline (speedup 1.0000x reference)
import functools

import jax
import jax.numpy as jnp
from jax import lax
from jax.experimental import pallas as pl
from jax.experimental.pallas import tpu as pltpu
from jax.experimental.pallas import tpu_sc as plsc

SMOOTHING = 0.1
CONFIDENCE = 1.0 - SMOOTHING

BC = 2048


def _sc_gather_build(B, C):
    info = plsc.get_sparse_core_info()
    nw = info.num_cores * info.num_subcores
    per_w = B // nw
    mesh = plsc.VectorSubcoreMesh(core_axis_name="c", subcore_axis_name="s")

    @functools.partial(
        pl.kernel,
        mesh=mesh,
        out_type=jax.ShapeDtypeStruct((B,), jnp.float32),
        scratch_types=[
            pltpu.VMEM((per_w,), jnp.int32),
            pltpu.VMEM((per_w,), jnp.float32),
            pltpu.SemaphoreType.DMA,
        ],
    )
    def gather_k(pred_flat_hbm, target_hbm, out_hbm, idx_v, vals_v, sem):
        wid = lax.axis_index("s") * info.num_cores + lax.axis_index("c")
        base = wid * per_w
        pltpu.sync_copy(target_hbm.at[pl.ds(base, per_w)], idx_v)
        for jj in range(per_w // 16):
            t = idx_v[pl.ds(jj * 16, 16)]
            rows = (base + jj * 16) + lax.iota(jnp.int32, 16)
            idx_v[pl.ds(jj * 16, 16)] = t + rows * C
        pltpu.async_copy(pred_flat_hbm.at[idx_v], vals_v, sem).wait()
        pltpu.sync_copy(vals_v, out_hbm.at[pl.ds(base, per_w)])

    return gather_k


def _tc_main_build(B, C):
    eps = SMOOTHING / (C - 1)
    nb = pl.cdiv(C, BC)
    tail = C - (nb - 1) * BC

    def body(pred_ref, vals_ref, out_ref, m_ref, s_ref, t_ref):
        j = pl.program_id(0)

        @pl.when(j == 0)
        def _():
            m_ref[...] = jnp.full_like(m_ref, -jnp.inf)
            s_ref[...] = jnp.zeros_like(s_ref)
            t_ref[...] = jnp.zeros_like(t_ref)

        x = pred_ref[...]

        def accum(xm, x0):
            m_old = m_ref[...]
            bm = jnp.max(xm, axis=1, keepdims=True)
            m_new = jnp.maximum(m_old, bm)
            e = jnp.exp(xm - m_new)
            s_ref[...] = s_ref[...] * jnp.exp(m_old - m_new) + jnp.sum(
                e, axis=1, keepdims=True
            )
            t_ref[...] += jnp.sum(x0, axis=1, keepdims=True)
            m_ref[...] = m_new

        @pl.when(j < nb - 1)
        def _():
            accum(x, x)

        @pl.when(j == nb - 1)
        def _():
            lanes = lax.broadcasted_iota(jnp.int32, x.shape, 1)
            mask = lanes < tail
            accum(
                jnp.where(mask, x, -jnp.inf),
                jnp.where(mask, x, 0.0),
            )
            lse = m_ref[...] + jnp.log(s_ref[...])
            loss = eps * (C * lse - t_ref[...]) + (CONFIDENCE - eps) * (
                lse - vals_ref[...]
            )
            out_ref[0, 0] = jnp.sum(loss) * (1.0 / B)

    return pl.pallas_call(
        body,
        grid=(nb,),
        in_specs=[
            pl.BlockSpec((B, BC), lambda j: (0, j)),
            pl.BlockSpec((B, 1), lambda j: (0, 0)),
        ],
        out_specs=pl.BlockSpec((1, 1), lambda j: (0, 0)),
        out_shape=jax.ShapeDtypeStruct((1, 1), jnp.float32),
        scratch_shapes=[
            pltpu.VMEM((B, 1), jnp.float32),
            pltpu.VMEM((B, 1), jnp.float32),
            pltpu.VMEM((B, 1), jnp.float32),
        ],
        compiler_params=pltpu.CompilerParams(
            dimension_semantics=("arbitrary",),
        ),
    )


def kernel(pred, target):
    B, C = pred.shape
    gather = _sc_gather_build(B, C)
    vals = gather(pred.reshape(-1), target.astype(jnp.int32))
    main = _tc_main_build(B, C)
    out = main(pred, vals.reshape(B, 1))
    return out[0, 0]


# speedup vs baseline: 1.2685x; 1.2685x over previous
import functools

import jax
import jax.numpy as jnp
from jax import lax
from jax.experimental import pallas as pl
from jax.experimental.pallas import tpu as pltpu
from jax.experimental.pallas import tpu_sc as plsc

SMOOTHING = 0.1
CONFIDENCE = 1.0 - SMOOTHING

BC = 2048


def _sc_gather_build(B, C):
    info = plsc.get_sparse_core_info()
    nw = info.num_cores * info.num_subcores
    per_w = B // nw
    mesh = plsc.VectorSubcoreMesh(core_axis_name="c", subcore_axis_name="s")

    @functools.partial(
        pl.kernel,
        mesh=mesh,
        out_type=jax.ShapeDtypeStruct((B,), jnp.float32),
        scratch_types=[
            pltpu.VMEM((per_w,), jnp.int32),
            pltpu.VMEM((per_w,), jnp.float32),
            pltpu.SemaphoreType.DMA,
        ],
    )
    def gather_k(pred_flat_hbm, target_hbm, out_hbm, idx_v, vals_v, sem):
        wid = lax.axis_index("s") * info.num_cores + lax.axis_index("c")
        base = wid * per_w
        pltpu.sync_copy(target_hbm.at[pl.ds(base, per_w)], idx_v)
        for jj in range(per_w // 16):
            t = idx_v[pl.ds(jj * 16, 16)]
            rows = (base + jj * 16) + lax.iota(jnp.int32, 16)
            idx_v[pl.ds(jj * 16, 16)] = t + rows * C
        pltpu.async_copy(pred_flat_hbm.at[idx_v], vals_v, sem).wait()
        pltpu.sync_copy(vals_v, out_hbm.at[pl.ds(base, per_w)])

    return gather_k


def _tc_main_build(B, C):
    eps = SMOOTHING / (C - 1)
    nb = pl.cdiv(C, BC)
    tail = C - (nb - 1) * BC

    def body(pred_ref, vals_ref, out_ref, m_ref, s_ref, t_ref):
        j = pl.program_id(0)

        @pl.when(j == 0)
        def _():
            m_ref[...] = jnp.full_like(m_ref, -jnp.inf)
            s_ref[...] = jnp.zeros_like(s_ref)
            t_ref[...] = jnp.zeros_like(t_ref)

        x = pred_ref[...]

        def accum(xm, x0):
            m_old = m_ref[...]
            bm = jnp.max(xm, axis=1, keepdims=True)
            m_new = jnp.maximum(m_old, bm)
            e = jnp.exp(xm - m_new)
            s_ref[...] = s_ref[...] * jnp.exp(m_old - m_new) + jnp.sum(
                e, axis=1, keepdims=True
            )
            t_ref[...] += jnp.sum(x0, axis=1, keepdims=True)
            m_ref[...] = m_new

        @pl.when(j < nb - 1)
        def _():
            accum(x, x)

        @pl.when(j == nb - 1)
        def _():
            lanes = lax.broadcasted_iota(jnp.int32, x.shape, 1)
            mask = lanes < tail
            accum(
                jnp.where(mask, x, -jnp.inf),
                jnp.where(mask, x, 0.0),
            )
            lse = m_ref[...] + jnp.log(s_ref[...])
            loss = eps * (C * lse - t_ref[...]) + (CONFIDENCE - eps) * (
                lse - vals_ref[...]
            )
            out_ref[...] = jnp.sum(loss, axis=(0, 1), keepdims=True) * (1.0 / B)

    return pl.pallas_call(
        body,
        grid=(nb,),
        in_specs=[
            pl.BlockSpec((B, BC), lambda j: (0, j)),
            pl.BlockSpec((B, 1), lambda j: (0, 0)),
        ],
        out_specs=pl.BlockSpec((1, 1), lambda j: (0, 0)),
        out_shape=jax.ShapeDtypeStruct((1, 1), jnp.float32),
        scratch_shapes=[
            pltpu.VMEM((B, 1), jnp.float32),
            pltpu.VMEM((B, 1), jnp.float32),
            pltpu.VMEM((B, 1), jnp.float32),
        ],
        compiler_params=pltpu.CompilerParams(
            dimension_semantics=("arbitrary",),
        ),
    )


def kernel(pred, target):
    B, C = pred.shape
    gather = _sc_gather_build(B, C)
    vals = gather(pred.reshape(-1), target.astype(jnp.int32))
    main = _tc_main_build(B, C)
    out = main(pred, vals.reshape(B, 1))
    return out[0, 0]
